# baseline (speedup 1.0000x reference)
import functools

import jax
import jax.numpy as jnp
from jax import lax
from jax.experimental import pallas as pl
from jax.experimental.pallas import tpu as pltpu
from jax.experimental.pallas import tpu_sc as plsc

_NC = 2
_NS = 16
_NW = _NC * _NS

_BB = 128
_L = 16


def _make_kernel(batch: int, fields: int, embed: int, padded: int):
    n_blk = batch // _BB
    blk_per_w = n_blk // _NW
    n_units = fields * blk_per_w
    c_tiles = embed // 8
    mesh = plsc.VectorSubcoreMesh(core_axis_name="c", subcore_axis_name="s")

    @functools.partial(
        pl.kernel,
        out_type=jax.ShapeDtypeStruct((fields * c_tiles * n_blk, 8 * _BB),
                                      jnp.float32),
        mesh=mesh,
        compiler_params=pltpu.CompilerParams(use_tc_tiling_on_sc=False,
                                             needs_layout_passes=False),
        scratch_types=[
            pltpu.VMEM((fields, blk_per_w * _BB), jnp.int32),
            pltpu.VMEM((_BB, padded), jnp.float32),
            pltpu.VMEM((_BB, padded), jnp.float32),
            pltpu.VMEM((embed * _BB,), jnp.float32),
            pltpu.VMEM((embed * _BB,), jnp.float32),
            pltpu.SemaphoreType.DMA,
            pltpu.SemaphoreType.DMA,
            pltpu.SemaphoreType.DMA,
            pltpu.SemaphoreType.DMA,
        ],
    )
    def body(tbl_hbm, xt_hbm, out_hbm, idx_v, dst0, dst1, st0, st1,
             gs0, gs1, os0, os1):
        wid = lax.axis_index("s") * _NC + lax.axis_index("c")
        bcol = wid * (blk_per_w * _BB)

        pltpu.sync_copy(
            xt_hbm.at[pl.ds(0, fields), pl.ds(bcol, blk_per_w * _BB)], idx_v)

        def fire(u, dst, sem):
            f = u // blk_per_w
            jl = lax.rem(u, blk_per_w)
            pltpu.async_copy(
                tbl_hbm.at[idx_v.at[f, pl.ds(jl * _BB, _BB)]], dst, sem)

        def drain(dst, sem):
            pltpu.make_async_copy(tbl_hbm.at[idx_v.at[0, pl.ds(0, _BB)]],
                                  dst, sem).wait()

        def transpose(dst, stage):
            def row(c, carry):
                base = (c // 8) * (8 * _BB) + lax.rem(c, 8) * _BB
                cols = jnp.full((_L,), c, jnp.int32)
                for k in range(_BB // _L):
                    rows = lax.iota(jnp.int32, _L) + k * _L
                    vals = plsc.load_gather(dst, [rows, cols])
                    stage[pl.ds(base + k * _L, _L)] = vals
                return carry
            lax.fori_loop(0, embed, row, 0)

        def fire_out(u, stage, sem):
            f = u // blk_per_w
            jl = lax.rem(u, blk_per_w)
            jb = wid * blk_per_w + jl
            for i in range(c_tiles):
                pltpu.async_copy(
                    stage.at[pl.ds(i * 8 * _BB, 8 * _BB)],
                    out_hbm.at[(f * c_tiles + i) * n_blk + jb], sem)

        def drain_out(stage, sem):
            for i in range(c_tiles):
                pltpu.make_async_copy(stage.at[pl.ds(i * 8 * _BB, 8 * _BB)],
                                      out_hbm.at[0], sem).wait()

        fire(0, dst0, gs0)

        def step(t, carry):
            ua = 2 * t
            fire(ua + 1, dst1, gs1)
            drain(dst0, gs0)

            @pl.when(t > 0)
            def _():
                drain_out(st0, os0)

            transpose(dst0, st0)
            fire_out(ua, st0, os0)

            @pl.when(t < n_units // 2 - 1)
            def _():
                fire(ua + 2, dst0, gs0)

            drain(dst1, gs1)

            @pl.when(t > 0)
            def _():
                drain_out(st1, os1)

            transpose(dst1, st1)
            fire_out(ua + 1, st1, os1)
            return carry

        lax.fori_loop(0, n_units // 2, step, 0)
        drain_out(st0, os0)
        drain_out(st1, os1)

    return body


def kernel(x, table):
    batch, fields = x.shape
    vocab, embed = table.shape
    padded = 2 * embed
    tbl128 = jnp.pad(table, ((0, 0), (0, padded - embed)))
    xt = x.T.astype(jnp.int32)
    n_blk = batch // _BB
    out5 = _make_kernel(batch, fields, embed, padded)(tbl128, xt)
    out = (out5.reshape(fields, embed // 8, n_blk, 8, _BB)
           .transpose(2, 4, 0, 1, 3)
           .reshape(batch, fields, embed))
    return out


# speedup vs baseline: 1.3149x; 1.3149x over previous
import functools

import jax
import jax.numpy as jnp
from jax import lax
from jax.experimental import pallas as pl
from jax.experimental.pallas import tpu as pltpu
from jax.experimental.pallas import tpu_sc as plsc

_NC = 2
_NS = 16
_NW = _NC * _NS

_BB = 128
_L = 16


def _make_kernel(batch: int, fields: int, embed: int, padded: int):
    n_blk = batch // _BB
    blk_per_w = n_blk // _NW
    n_units = fields * blk_per_w
    c_tiles = embed // 8
    mesh = plsc.VectorSubcoreMesh(core_axis_name="c", subcore_axis_name="s")

    @functools.partial(
        pl.kernel,
        out_type=jax.ShapeDtypeStruct((fields * c_tiles * n_blk, 8 * _BB),
                                      jnp.float32),
        mesh=mesh,
        compiler_params=pltpu.CompilerParams(use_tc_tiling_on_sc=False,
                                             needs_layout_passes=False),
        scratch_types=[
            pltpu.VMEM((fields, blk_per_w * _BB), jnp.int32),
            pltpu.VMEM((_BB, padded), jnp.float32),
            pltpu.VMEM((_BB, padded), jnp.float32),
            pltpu.VMEM((embed * _BB,), jnp.float32),
            pltpu.VMEM((embed * _BB,), jnp.float32),
            pltpu.SemaphoreType.DMA,
            pltpu.SemaphoreType.DMA,
            pltpu.SemaphoreType.DMA,
            pltpu.SemaphoreType.DMA,
        ],
    )
    def body(tbl_hbm, xt_hbm, out_hbm, idx_v, dst0, dst1, st0, st1,
             gs0, gs1, os0, os1):
        wid = lax.axis_index("s") * _NC + lax.axis_index("c")
        bcol = wid * (blk_per_w * _BB)

        pltpu.sync_copy(
            xt_hbm.at[pl.ds(0, fields), pl.ds(bcol, blk_per_w * _BB)], idx_v)

        def fire(u, dst, sem):
            f = u // blk_per_w
            jl = lax.rem(u, blk_per_w)
            pltpu.async_copy(
                tbl_hbm.at[idx_v.at[f, pl.ds(jl * _BB, _BB)]], dst, sem)

        def drain(dst, sem):
            pltpu.make_async_copy(tbl_hbm.at[idx_v.at[0, pl.ds(0, _BB)]],
                                  dst, sem).wait()

        def transpose(dst, stage):
            @plsc.parallel_loop(0, embed, 2, unroll=4)
            def _(c):
                for cc in range(2):
                    base = ((c + cc) // 8) * (8 * _BB) + lax.rem(c + cc, 8) * _BB
                    cols = jnp.full((_L,), c + cc, jnp.int32)
                    for k in range(_BB // _L):
                        rows = lax.iota(jnp.int32, _L) + k * _L
                        vals = plsc.load_gather(dst, [rows, cols])
                        stage[pl.ds(base + k * _L, _L)] = vals

        def fire_out(u, stage, sem):
            f = u // blk_per_w
            jl = lax.rem(u, blk_per_w)
            jb = wid * blk_per_w + jl
            for i in range(c_tiles):
                pltpu.async_copy(
                    stage.at[pl.ds(i * 8 * _BB, 8 * _BB)],
                    out_hbm.at[(f * c_tiles + i) * n_blk + jb], sem)

        def drain_out(stage, sem):
            for i in range(c_tiles):
                pltpu.make_async_copy(stage.at[pl.ds(i * 8 * _BB, 8 * _BB)],
                                      out_hbm.at[0], sem).wait()

        fire(0, dst0, gs0)

        def step(t, carry):
            ua = 2 * t
            fire(ua + 1, dst1, gs1)
            drain(dst0, gs0)

            @pl.when(t > 0)
            def _():
                drain_out(st0, os0)

            transpose(dst0, st0)
            fire_out(ua, st0, os0)

            @pl.when(t < n_units // 2 - 1)
            def _():
                fire(ua + 2, dst0, gs0)

            drain(dst1, gs1)

            @pl.when(t > 0)
            def _():
                drain_out(st1, os1)

            transpose(dst1, st1)
            fire_out(ua + 1, st1, os1)
            return carry

        lax.fori_loop(0, n_units // 2, step, 0)
        drain_out(st0, os0)
        drain_out(st1, os1)

    return body


def kernel(x, table):
    batch, fields = x.shape
    vocab, embed = table.shape
    padded = 2 * embed
    tbl128 = jnp.pad(table, ((0, 0), (0, padded - embed)))
    xt = x.T.astype(jnp.int32)
    n_blk = batch // _BB
    out5 = _make_kernel(batch, fields, embed, padded)(tbl128, xt)
    out = (out5.reshape(fields, embed // 8, n_blk, 8, _BB)
           .transpose(2, 4, 0, 1, 3)
           .reshape(batch, fields, embed))
    return out


# speedup vs baseline: 1.3155x; 1.0004x over previous
import functools

import jax
import jax.numpy as jnp
from jax import lax
from jax.experimental import pallas as pl
from jax.experimental.pallas import tpu as pltpu
from jax.experimental.pallas import tpu_sc as plsc

_NC = 2
_NS = 16
_NW = _NC * _NS

_BB = 128
_L = 16


def _make_kernel(batch: int, fields: int, embed: int, padded: int):
    n_blk = batch // _BB
    blk_per_w = n_blk // _NW
    n_units = fields * blk_per_w
    c_tiles = embed // 8
    mesh = plsc.VectorSubcoreMesh(core_axis_name="c", subcore_axis_name="s")

    @functools.partial(
        pl.kernel,
        out_type=jax.ShapeDtypeStruct((fields * c_tiles * n_blk, 8 * _BB),
                                      jnp.float32),
        mesh=mesh,
        compiler_params=pltpu.CompilerParams(use_tc_tiling_on_sc=False,
                                             needs_layout_passes=False),
        scratch_types=[
            pltpu.VMEM((fields, blk_per_w * _BB), jnp.int32),
            pltpu.VMEM((_BB, padded), jnp.float32),
            pltpu.VMEM((_BB, padded), jnp.float32),
            pltpu.VMEM((embed * _BB,), jnp.float32),
            pltpu.VMEM((embed * _BB,), jnp.float32),
            pltpu.SemaphoreType.DMA,
            pltpu.SemaphoreType.DMA,
            pltpu.SemaphoreType.DMA,
            pltpu.SemaphoreType.DMA,
        ],
    )
    def body(tbl_hbm, xt_hbm, out_hbm, idx_v, dst0, dst1, st0, st1,
             gs0, gs1, os0, os1):
        wid = lax.axis_index("s") * _NC + lax.axis_index("c")
        bcol = wid * (blk_per_w * _BB)

        pltpu.sync_copy(
            xt_hbm.at[pl.ds(0, fields), pl.ds(bcol, blk_per_w * _BB)], idx_v)

        def fire(u, dst, sem):
            f = u // blk_per_w
            jl = lax.rem(u, blk_per_w)
            pltpu.async_copy(
                tbl_hbm.at[idx_v.at[f, pl.ds(jl * _BB, _BB)]], dst, sem)

        def drain(dst, sem):
            pltpu.make_async_copy(tbl_hbm.at[idx_v.at[0, pl.ds(0, _BB)]],
                                  dst, sem).wait()

        row_vecs = [lax.iota(jnp.int32, _L) + k * _L for k in range(_BB // _L)]

        def transpose(dst, stage):
            @plsc.parallel_loop(0, embed, 2, unroll=4)
            def _(c):
                for cc in range(2):
                    base = ((c + cc) // 8) * (8 * _BB) + lax.rem(c + cc, 8) * _BB
                    cols = jnp.full((_L,), c + cc, jnp.int32)
                    for k in range(_BB // _L):
                        vals = plsc.load_gather(dst, [row_vecs[k], cols])
                        stage[pl.ds(base + k * _L, _L)] = vals

        def fire_out(u, stage, sem):
            f = u // blk_per_w
            jl = lax.rem(u, blk_per_w)
            jb = wid * blk_per_w + jl
            for i in range(c_tiles):
                pltpu.async_copy(
                    stage.at[pl.ds(i * 8 * _BB, 8 * _BB)],
                    out_hbm.at[(f * c_tiles + i) * n_blk + jb], sem)

        def drain_out(stage, sem):
            for i in range(c_tiles):
                pltpu.make_async_copy(stage.at[pl.ds(i * 8 * _BB, 8 * _BB)],
                                      out_hbm.at[0], sem).wait()

        fire(0, dst0, gs0)

        def step(t, carry):
            ua = 2 * t
            fire(ua + 1, dst1, gs1)
            drain(dst0, gs0)

            @pl.when(t > 0)
            def _():
                drain_out(st0, os0)

            transpose(dst0, st0)
            fire_out(ua, st0, os0)

            @pl.when(t < n_units // 2 - 1)
            def _():
                fire(ua + 2, dst0, gs0)

            drain(dst1, gs1)

            @pl.when(t > 0)
            def _():
                drain_out(st1, os1)

            transpose(dst1, st1)
            fire_out(ua + 1, st1, os1)
            return carry

        lax.fori_loop(0, n_units // 2, step, 0)
        drain_out(st0, os0)
        drain_out(st1, os1)

    return body


def kernel(x, table):
    batch, fields = x.shape
    vocab, embed = table.shape
    padded = 2 * embed
    tbl128 = jnp.pad(table, ((0, 0), (0, padded - embed)))
    xt = x.T.astype(jnp.int32)
    n_blk = batch // _BB
    out5 = _make_kernel(batch, fields, embed, padded)(tbl128, xt)
    out = (out5.reshape(fields, embed // 8, n_blk, 8, _BB)
           .transpose(2, 4, 0, 1, 3)
           .reshape(batch, fields, embed))
    return out
